# Initial kernel scaffold; baseline (speedup 1.0000x reference)
import jax
import jax.numpy as jnp
from jax.experimental import pallas as pl


def kernel(node_emb, W1, root1, b1, W2, root2, b2, edge_index, edge_type):
    raise NotImplementedError("write your pallas kernel here")


# speedup vs baseline: 32.1468x; 32.1468x over previous
import functools

import jax
import jax.numpy as jnp
from jax import lax
from jax.experimental import pallas as pl
from jax.experimental.pallas import tpu as pltpu
from jax.experimental.pallas import tpu_sc as plsc

N, R, H, E, D = 100000, 16, 32, 1600000, 16
NR = N * R
NC, NS, L = 2, 16, 16
SB = 80
CH = 2000
NPT = N // NS
EPT = E // NS
_mesh = plsc.VectorSubcoreMesh(
    core_axis_name="c", subcore_axis_name="s", num_cores=NC, num_subcores=NS)
_sc_params = pltpu.CompilerParams(use_tc_tiling_on_sc=False)


def _keys80(ab, bb, j, kb):
  def vec(v, _):
    sl = pl.ds(j * SB + v * L, L)
    k = ab[sl] * R + bb[sl]
    kb[pl.ds(v * L, L)] = jnp.minimum(jnp.maximum(k, 0), NR - 1)
    return 0
  lax.fori_loop(0, SB // L, vec, 0)


@functools.partial(
    pl.kernel,
    out_type=(
        jax.ShapeDtypeStruct((E,), jnp.int32),
        jax.ShapeDtypeStruct((NR,), jnp.float32),
    ),
    mesh=_mesh,
    compiler_params=_sc_params,
    scratch_types=[
        pltpu.VMEM((CH,), jnp.int32),
        pltpu.VMEM((CH,), jnp.int32),
        pltpu.VMEM((CH,), jnp.int32),
        pltpu.VMEM((CH,), jnp.int32),
        pltpu.VMEM((SB,), jnp.int32),
        pltpu.VMEM((SB,), jnp.float32),
        pltpu.VMEM((CH,), jnp.float32),
        pltpu.VMEM_SHARED((NR,), jnp.float32),
    ],
)
def _pre1(src_h, dst_h, et_h, gk_h, inv_hbm, srcb, dstb, typeb, gkb, k2b,
          ones, cbuf, cnt_sh):
  c = lax.axis_index("c")
  t = lax.axis_index("s")

  @pl.when(c == 0)
  def _():
    def zv(u, _):
      cbuf[pl.ds(u * L, L)] = jnp.zeros((L,), jnp.float32)
      return 0
    lax.fori_loop(0, CH // L, zv, 0)

    def z(i, _):
      pltpu.sync_copy(cbuf, cnt_sh.at[pl.ds(t * (NR // NS) + i * CH, CH)])
      return 0
    lax.fori_loop(0, NR // NS // CH, z, 0)
    for u in range(SB // L):
      ones[pl.ds(u * L, L)] = jnp.ones((L,), jnp.float32)
    plsc.subcore_barrier()

    def chunk(ch, _):
      e0 = t * EPT + ch * CH
      pltpu.sync_copy(src_h.at[pl.ds(e0, CH)], srcb)
      pltpu.sync_copy(dst_h.at[pl.ds(e0, CH)], dstb)
      pltpu.sync_copy(et_h.at[pl.ds(e0, CH)], typeb)

      def gv(v, _):
        sl = pl.ds(v * L, L)
        g = srcb[sl] * R + typeb[sl]
        gkb[sl] = jnp.minimum(jnp.maximum(g, 0), NR - 1)
        return 0
      lax.fori_loop(0, CH // L, gv, 0)
      pltpu.sync_copy(gkb, gk_h.at[pl.ds(e0, CH)])

      def sub(j, _):
        _keys80(dstb, typeb, j, k2b)
        pltpu.sync_copy(ones, cnt_sh.at[k2b], add=True)
        return 0
      lax.fori_loop(0, CH // SB, sub, 0)
      return 0
    lax.fori_loop(0, EPT // CH, chunk, 0)
    plsc.subcore_barrier()

    def ichunk(i, _):
      base = t * (NR // NS) + i * CH
      pltpu.sync_copy(cnt_sh.at[pl.ds(base, CH)], cbuf)

      def vec(v, _):
        sl = pl.ds(v * L, L)
        cbuf[sl] = 1.0 / jnp.maximum(cbuf[sl], 1.0)
        return 0
      lax.fori_loop(0, CH // L, vec, 0)
      pltpu.sync_copy(cbuf, inv_hbm.at[pl.ds(base, CH)])
      return 0
    lax.fori_loop(0, NR // NS // CH, ichunk, 0)


@functools.partial(
    pl.kernel,
    out_type=jax.ShapeDtypeStruct((E,), jnp.float32),
    mesh=_mesh,
    compiler_params=_sc_params,
    scratch_types=[
        pltpu.VMEM((CH,), jnp.int32),
        pltpu.VMEM((CH,), jnp.int32),
        pltpu.VMEM((SB,), jnp.int32),
        pltpu.VMEM((SB,), jnp.float32),
        pltpu.VMEM((CH,), jnp.float32),
        pltpu.SemaphoreType.DMA,
    ],
)
def _pre2(inv_hbm, dst_h, et_h, s_h, dstb, typeb, k2b, sv80, sbuf, sem):
  w = lax.axis_index("s") * NC + lax.axis_index("c")
  ew = E // (NC * NS)

  def chunk(ch, _):
    e0 = w * ew + ch * CH
    pltpu.sync_copy(dst_h.at[pl.ds(e0, CH)], dstb)
    pltpu.sync_copy(et_h.at[pl.ds(e0, CH)], typeb)

    def sub(j, _):
      _keys80(dstb, typeb, j, k2b)
      pltpu.async_copy(inv_hbm.at[k2b], sv80, sem).wait()

      def cpv(v, _):
        sbuf[pl.ds(j * SB + v * L, L)] = sv80[pl.ds(v * L, L)]
        return 0
      lax.fori_loop(0, SB // L, cpv, 0)
      return 0
    lax.fori_loop(0, CH // SB, sub, 0)
    pltpu.sync_copy(sbuf, s_h.at[pl.ds(e0, CH)])
    return 0
  lax.fori_loop(0, ew // CH, chunk, 0)


@functools.partial(
    pl.kernel,
    out_type=jax.ShapeDtypeStruct((2, N, L), jnp.float32),
    mesh=_mesh,
    compiler_params=_sc_params,
    scratch_types=[
        pltpu.VMEM((CH,), jnp.int32),
        pltpu.VMEM((CH,), jnp.int32),
        pltpu.VMEM((CH,), jnp.float32),
        pltpu.VMEM((SB,), jnp.int32),
        pltpu.VMEM((SB,), jnp.int32),
        pltpu.VMEM((SB, L), jnp.float32),
        pltpu.VMEM((NPT // 50, L), jnp.float32),
        pltpu.VMEM((NPT // 50, L), jnp.float32),
        pltpu.VMEM_SHARED((N, L), jnp.float32),
        pltpu.SemaphoreType.DMA,
    ],
)
def _agg(ylo, yhi, gk_h, dst_h, s_h, z0, out, gkb, dstb, sb, gk80, d80, rows,
         accb, z0b, acc_sh, sem):
  c = lax.axis_index("c")
  t = lax.axis_index("s")
  cp_rows = NPT // 50

  def body(ytab, z0h, outh):
    def zr(i, _):
      accb[i] = jnp.zeros((L,), jnp.float32)
      return 0
    lax.fori_loop(0, cp_rows, zr, 0)

    def zc(i, _):
      pltpu.sync_copy(accb, acc_sh.at[pl.ds(t * NPT + i * cp_rows, cp_rows)])
      return 0
    lax.fori_loop(0, 50, zc, 0)
    plsc.subcore_barrier()

    def chunk(ch, _):
      e0 = t * EPT + ch * CH
      pltpu.sync_copy(gk_h.at[pl.ds(e0, CH)], gkb)
      pltpu.sync_copy(dst_h.at[pl.ds(e0, CH)], dstb)
      pltpu.sync_copy(s_h.at[pl.ds(e0, CH)], sb)

      def sub(j, _):
        def kv(v, _):
          sl = pl.ds(j * SB + v * L, L)
          g = gkb[sl]
          gk80[pl.ds(v * L, L)] = jnp.minimum(jnp.maximum(g, 0), NR - 1)
          d = dstb[sl]
          d80[pl.ds(v * L, L)] = jnp.minimum(jnp.maximum(d, 0), N - 1)
          return 0
        lax.fori_loop(0, SB // L, kv, 0)
        pltpu.async_copy(ytab.at[gk80], rows, sem).wait()

        def scale(v, _):
          sv = sb[pl.ds(j * SB + v * L, L)]
          for u in range(L):
            e = v * L + u
            rows[e] = rows[e] * sv[u]
          return 0
        lax.fori_loop(0, SB // L, scale, 0)
        pltpu.sync_copy(rows, acc_sh.at[d80], add=True)
        return 0
      lax.fori_loop(0, CH // SB, sub, 0)
      return 0
    lax.fori_loop(0, EPT // CH, chunk, 0)
    plsc.subcore_barrier()

    def cp(i, _):
      base = t * NPT + i * cp_rows
      pltpu.sync_copy(acc_sh.at[pl.ds(base, cp_rows)], accb)
      pltpu.sync_copy(z0h.at[pl.ds(base, cp_rows)], z0b)

      def add(r_, _):
        accb[r_] = accb[r_] + z0b[r_]
        return 0
      lax.fori_loop(0, cp_rows, add, 0)
      pltpu.sync_copy(accb, outh.at[pl.ds(base, cp_rows)])
      return 0
    lax.fori_loop(0, 50, cp, 0)

  @pl.when(c == 0)
  def _():
    body(ylo, z0.at[0], out.at[0])

  @pl.when(c == 1)
  def _():
    body(yhi, z0.at[1], out.at[1])


_BLK = 2000


def _dense_body(apply_relu, x_ref, wlo_ref, whi_ref, root_ref, b_ref,
                ylo_ref, yhi_ref, z0_ref):
  x = x_ref[...]
  if apply_relu:
    x = jnp.maximum(jnp.concatenate([x[0], x[1]], axis=1), 0.0)
  ylo_ref[...] = jnp.dot(x, wlo_ref[...], preferred_element_type=jnp.float32)
  yhi_ref[...] = jnp.dot(x, whi_ref[...], preferred_element_type=jnp.float32)
  z0 = jnp.dot(x, root_ref[...], preferred_element_type=jnp.float32) + b_ref[...]
  z0_ref[0] = z0[:, :L]
  z0_ref[1] = z0[:, L:]


def _dense(x, wlo, whi, root, b, apply_relu):
  din = root.shape[0]
  if apply_relu:
    in_spec0 = pl.BlockSpec((2, _BLK, L), lambda i: (0, i, 0))
  else:
    in_spec0 = pl.BlockSpec((_BLK, din), lambda i: (i, 0))
  return pl.pallas_call(
      functools.partial(_dense_body, apply_relu),
      grid=(N // _BLK,),
      in_specs=[
          in_spec0,
          pl.BlockSpec((din, R * L), lambda i: (0, 0)),
          pl.BlockSpec((din, R * L), lambda i: (0, 0)),
          pl.BlockSpec((din, H), lambda i: (0, 0)),
          pl.BlockSpec((1, H), lambda i: (0, 0)),
      ],
      out_specs=[
          pl.BlockSpec((_BLK, R * L), lambda i: (i, 0)),
          pl.BlockSpec((_BLK, R * L), lambda i: (i, 0)),
          pl.BlockSpec((2, _BLK, L), lambda i: (0, i, 0)),
      ],
      out_shape=[
          jax.ShapeDtypeStruct((N, R * L), jnp.float32),
          jax.ShapeDtypeStruct((N, R * L), jnp.float32),
          jax.ShapeDtypeStruct((2, N, L), jnp.float32),
      ],
  )(x, wlo, whi, root, b)


def kernel(node_emb, W1, root1, b1, W2, root2, b2, edge_index, edge_type):
  src_h = edge_index[0]
  dst_h = edge_index[1]

  gk_h, inv = _pre1(src_h, dst_h, edge_type)
  s_h = _pre2(inv, dst_h, edge_type)

  wlo1 = jnp.transpose(W1[:, :, :L], (1, 0, 2)).reshape(D, R * L)
  whi1 = jnp.transpose(W1[:, :, L:], (1, 0, 2)).reshape(D, R * L)
  wlo2 = jnp.transpose(W2[:, :, :L], (1, 0, 2)).reshape(H, R * L)
  whi2 = jnp.transpose(W2[:, :, L:], (1, 0, 2)).reshape(H, R * L)

  ylo1, yhi1, z01 = _dense(node_emb, wlo1, whi1, root1, b1.reshape(1, H),
                           apply_relu=False)
  z1 = _agg(ylo1.reshape(NR, L), yhi1.reshape(NR, L), gk_h, dst_h, s_h, z01)
  ylo2, yhi2, z02 = _dense(z1, wlo2, whi2, root2, b2.reshape(1, H),
                           apply_relu=True)
  z2 = _agg(ylo2.reshape(NR, L), yhi2.reshape(NR, L), gk_h, dst_h, s_h, z02)
  return jnp.concatenate([z2[0], z2[1]], axis=1)


# speedup vs baseline: 40.1532x; 1.2491x over previous
import functools

import jax
import jax.numpy as jnp
from jax import lax
from jax.experimental import pallas as pl
from jax.experimental.pallas import tpu as pltpu
from jax.experimental.pallas import tpu_sc as plsc

N, R, H, E, D = 100000, 16, 32, 1600000, 16
NR = N * R
NC, NS, L = 2, 16, 16
SB = 80
CH = 2000
NPT = N // NS
EPT = E // NS
_mesh = plsc.VectorSubcoreMesh(
    core_axis_name="c", subcore_axis_name="s", num_cores=NC, num_subcores=NS)
_sc_params = pltpu.CompilerParams(use_tc_tiling_on_sc=False)


def _keys80(ab, bb, j, kb):
  def vec(v, _):
    sl = pl.ds(j * SB + v * L, L)
    k = ab[sl] * R + bb[sl]
    kb[pl.ds(v * L, L)] = jnp.minimum(jnp.maximum(k, 0), NR - 1)
    return 0
  lax.fori_loop(0, SB // L, vec, 0)


@functools.partial(
    pl.kernel,
    out_type=(
        jax.ShapeDtypeStruct((E,), jnp.int32),
        jax.ShapeDtypeStruct((NR,), jnp.float32),
    ),
    mesh=_mesh,
    compiler_params=_sc_params,
    scratch_types=[
        pltpu.VMEM((CH,), jnp.int32),
        pltpu.VMEM((CH,), jnp.int32),
        pltpu.VMEM((CH,), jnp.int32),
        pltpu.VMEM((CH,), jnp.int32),
        pltpu.VMEM((SB,), jnp.int32),
        pltpu.VMEM((SB,), jnp.float32),
        pltpu.VMEM((CH,), jnp.float32),
        pltpu.VMEM_SHARED((NR,), jnp.float32),
    ],
)
def _pre1(src_h, dst_h, et_h, gk_h, inv_hbm, srcb, dstb, typeb, gkb, k2b,
          ones, cbuf, cnt_sh):
  c = lax.axis_index("c")
  t = lax.axis_index("s")

  @pl.when(c == 0)
  def _():
    def zv(u, _):
      cbuf[pl.ds(u * L, L)] = jnp.zeros((L,), jnp.float32)
      return 0
    lax.fori_loop(0, CH // L, zv, 0)

    def z(i, _):
      pltpu.sync_copy(cbuf, cnt_sh.at[pl.ds(t * (NR // NS) + i * CH, CH)])
      return 0
    lax.fori_loop(0, NR // NS // CH, z, 0)
    for u in range(SB // L):
      ones[pl.ds(u * L, L)] = jnp.ones((L,), jnp.float32)
    plsc.subcore_barrier()

    def chunk(ch, _):
      e0 = t * EPT + ch * CH
      pltpu.sync_copy(src_h.at[pl.ds(e0, CH)], srcb)
      pltpu.sync_copy(dst_h.at[pl.ds(e0, CH)], dstb)
      pltpu.sync_copy(et_h.at[pl.ds(e0, CH)], typeb)

      def gv(v, _):
        sl = pl.ds(v * L, L)
        g = srcb[sl] * R + typeb[sl]
        gkb[sl] = jnp.minimum(jnp.maximum(g, 0), NR - 1)
        return 0
      lax.fori_loop(0, CH // L, gv, 0)
      pltpu.sync_copy(gkb, gk_h.at[pl.ds(e0, CH)])

      def sub(j, _):
        _keys80(dstb, typeb, j, k2b)
        pltpu.sync_copy(ones, cnt_sh.at[k2b], add=True)
        return 0
      lax.fori_loop(0, CH // SB, sub, 0)
      return 0
    lax.fori_loop(0, EPT // CH, chunk, 0)
    plsc.subcore_barrier()

    def ichunk(i, _):
      base = t * (NR // NS) + i * CH
      pltpu.sync_copy(cnt_sh.at[pl.ds(base, CH)], cbuf)

      def vec(v, _):
        sl = pl.ds(v * L, L)
        cbuf[sl] = 1.0 / jnp.maximum(cbuf[sl], 1.0)
        return 0
      lax.fori_loop(0, CH // L, vec, 0)
      pltpu.sync_copy(cbuf, inv_hbm.at[pl.ds(base, CH)])
      return 0
    lax.fori_loop(0, NR // NS // CH, ichunk, 0)


@functools.partial(
    pl.kernel,
    out_type=jax.ShapeDtypeStruct((E,), jnp.float32),
    mesh=_mesh,
    compiler_params=_sc_params,
    scratch_types=[
        pltpu.VMEM((CH,), jnp.int32),
        pltpu.VMEM((CH,), jnp.int32),
        pltpu.VMEM((SB,), jnp.int32),
        pltpu.VMEM((SB,), jnp.float32),
        pltpu.VMEM((CH,), jnp.float32),
        pltpu.SemaphoreType.DMA,
    ],
)
def _pre2(inv_hbm, dst_h, et_h, s_h, dstb, typeb, k2b, sv80, sbuf, sem):
  w = lax.axis_index("s") * NC + lax.axis_index("c")
  ew = E // (NC * NS)

  def chunk(ch, _):
    e0 = w * ew + ch * CH
    pltpu.sync_copy(dst_h.at[pl.ds(e0, CH)], dstb)
    pltpu.sync_copy(et_h.at[pl.ds(e0, CH)], typeb)

    def sub(j, _):
      _keys80(dstb, typeb, j, k2b)
      pltpu.async_copy(inv_hbm.at[k2b], sv80, sem).wait()

      def cpv(v, _):
        sbuf[pl.ds(j * SB + v * L, L)] = sv80[pl.ds(v * L, L)]
        return 0
      lax.fori_loop(0, SB // L, cpv, 0)
      return 0
    lax.fori_loop(0, CH // SB, sub, 0)
    pltpu.sync_copy(sbuf, s_h.at[pl.ds(e0, CH)])
    return 0
  lax.fori_loop(0, ew // CH, chunk, 0)


@functools.partial(
    pl.kernel,
    out_type=jax.ShapeDtypeStruct((2, N, L), jnp.float32),
    mesh=_mesh,
    compiler_params=_sc_params,
    scratch_types=[
        pltpu.VMEM((CH,), jnp.int32),
        pltpu.VMEM((CH,), jnp.int32),
        pltpu.VMEM((CH,), jnp.float32),
        pltpu.VMEM((SB,), jnp.int32),
        pltpu.VMEM((SB,), jnp.int32),
        pltpu.VMEM((SB,), jnp.int32),
        pltpu.VMEM((SB,), jnp.int32),
        pltpu.VMEM((SB, L), jnp.float32),
        pltpu.VMEM((SB, L), jnp.float32),
        pltpu.VMEM((NPT // 50, L), jnp.float32),
        pltpu.VMEM((NPT // 50, L), jnp.float32),
        pltpu.VMEM_SHARED((N, L), jnp.float32),
        pltpu.SemaphoreType.DMA,
        pltpu.SemaphoreType.DMA,
    ],
)
def _agg(ylo, yhi, gk_h, dst_h, s_h, z0, out, gkb, dstb, sb, gk80, d80,
         gk80b, d80b, rows, rowsb, accb, z0b, acc_sh, sem, semb):
  c = lax.axis_index("c")
  t = lax.axis_index("s")
  cp_rows = NPT // 50

  def body(ytab, z0h, outh):
    def zr(i, _):
      accb[i] = jnp.zeros((L,), jnp.float32)
      return 0
    lax.fori_loop(0, cp_rows, zr, 0)

    def zc(i, _):
      pltpu.sync_copy(accb, acc_sh.at[pl.ds(t * NPT + i * cp_rows, cp_rows)])
      return 0
    lax.fori_loop(0, 50, zc, 0)
    plsc.subcore_barrier()

    def chunk(ch, _):
      e0 = t * EPT + ch * CH
      pltpu.sync_copy(gk_h.at[pl.ds(e0, CH)], gkb)
      pltpu.sync_copy(dst_h.at[pl.ds(e0, CH)], dstb)
      pltpu.sync_copy(s_h.at[pl.ds(e0, CH)], sb)

      def keys_for(j, kg, kd):
        def kv(v, _):
          sl = pl.ds(j * SB + v * L, L)
          g = gkb[sl]
          kg[pl.ds(v * L, L)] = jnp.minimum(jnp.maximum(g, 0), NR - 1)
          d = dstb[sl]
          kd[pl.ds(v * L, L)] = jnp.minimum(jnp.maximum(d, 0), N - 1)
          return 0
        lax.fori_loop(0, SB // L, kv, 0)

      def scale_rows(j, rws):
        def scale(v, _):
          sv = sb[pl.ds(j * SB + v * L, L)]
          for u in range(L):
            e = v * L + u
            rws[e] = rws[e] * sv[u]
          return 0
        lax.fori_loop(0, SB // L, scale, 0)

      def pair(p, _):
        ja = 2 * p
        jb = 2 * p + 1
        keys_for(ja, gk80, d80)
        ca = pltpu.async_copy(ytab.at[gk80], rows, sem)
        keys_for(jb, gk80b, d80b)
        cb = pltpu.async_copy(ytab.at[gk80b], rowsb, semb)
        ca.wait()
        scale_rows(ja, rows)
        pltpu.sync_copy(rows, acc_sh.at[d80], add=True)
        cb.wait()
        scale_rows(jb, rowsb)
        pltpu.sync_copy(rowsb, acc_sh.at[d80b], add=True)
        return 0
      lax.fori_loop(0, (CH // SB) // 2, pair, 0)
      keys_for(CH // SB - 1, gk80, d80)
      pltpu.async_copy(ytab.at[gk80], rows, sem).wait()
      scale_rows(CH // SB - 1, rows)
      pltpu.sync_copy(rows, acc_sh.at[d80], add=True)
      return 0
    lax.fori_loop(0, EPT // CH, chunk, 0)
    plsc.subcore_barrier()

    def cp(i, _):
      base = t * NPT + i * cp_rows
      pltpu.sync_copy(acc_sh.at[pl.ds(base, cp_rows)], accb)
      pltpu.sync_copy(z0h.at[pl.ds(base, cp_rows)], z0b)

      def add(r_, _):
        accb[r_] = accb[r_] + z0b[r_]
        return 0
      lax.fori_loop(0, cp_rows, add, 0)
      pltpu.sync_copy(accb, outh.at[pl.ds(base, cp_rows)])
      return 0
    lax.fori_loop(0, 50, cp, 0)

  @pl.when(c == 0)
  def _():
    body(ylo, z0.at[0], out.at[0])

  @pl.when(c == 1)
  def _():
    body(yhi, z0.at[1], out.at[1])


_BLK = 2000


def _dense_body(apply_relu, x_ref, wlo_ref, whi_ref, root_ref, b_ref,
                ylo_ref, yhi_ref, z0_ref):
  x = x_ref[...]
  if apply_relu:
    x = jnp.maximum(jnp.concatenate([x[0], x[1]], axis=1), 0.0)
  ylo_ref[...] = jnp.dot(x, wlo_ref[...], preferred_element_type=jnp.float32)
  yhi_ref[...] = jnp.dot(x, whi_ref[...], preferred_element_type=jnp.float32)
  z0 = jnp.dot(x, root_ref[...], preferred_element_type=jnp.float32) + b_ref[...]
  z0_ref[0] = z0[:, :L]
  z0_ref[1] = z0[:, L:]


def _dense(x, wlo, whi, root, b, apply_relu):
  din = root.shape[0]
  if apply_relu:
    in_spec0 = pl.BlockSpec((2, _BLK, L), lambda i: (0, i, 0))
  else:
    in_spec0 = pl.BlockSpec((_BLK, din), lambda i: (i, 0))
  return pl.pallas_call(
      functools.partial(_dense_body, apply_relu),
      grid=(N // _BLK,),
      in_specs=[
          in_spec0,
          pl.BlockSpec((din, R * L), lambda i: (0, 0)),
          pl.BlockSpec((din, R * L), lambda i: (0, 0)),
          pl.BlockSpec((din, H), lambda i: (0, 0)),
          pl.BlockSpec((1, H), lambda i: (0, 0)),
      ],
      out_specs=[
          pl.BlockSpec((_BLK, R * L), lambda i: (i, 0)),
          pl.BlockSpec((_BLK, R * L), lambda i: (i, 0)),
          pl.BlockSpec((2, _BLK, L), lambda i: (0, i, 0)),
      ],
      out_shape=[
          jax.ShapeDtypeStruct((N, R * L), jnp.float32),
          jax.ShapeDtypeStruct((N, R * L), jnp.float32),
          jax.ShapeDtypeStruct((2, N, L), jnp.float32),
      ],
  )(x, wlo, whi, root, b)


def kernel(node_emb, W1, root1, b1, W2, root2, b2, edge_index, edge_type):
  src_h = edge_index[0]
  dst_h = edge_index[1]

  gk_h, inv = _pre1(src_h, dst_h, edge_type)
  s_h = _pre2(inv, dst_h, edge_type)

  wlo1 = jnp.transpose(W1[:, :, :L], (1, 0, 2)).reshape(D, R * L)
  whi1 = jnp.transpose(W1[:, :, L:], (1, 0, 2)).reshape(D, R * L)
  wlo2 = jnp.transpose(W2[:, :, :L], (1, 0, 2)).reshape(H, R * L)
  whi2 = jnp.transpose(W2[:, :, L:], (1, 0, 2)).reshape(H, R * L)

  ylo1, yhi1, z01 = _dense(node_emb, wlo1, whi1, root1, b1.reshape(1, H),
                           apply_relu=False)
  z1 = _agg(ylo1.reshape(NR, L), yhi1.reshape(NR, L), gk_h, dst_h, s_h, z01)
  ylo2, yhi2, z02 = _dense(z1, wlo2, whi2, root2, b2.reshape(1, H),
                           apply_relu=True)
  z2 = _agg(ylo2.reshape(NR, L), yhi2.reshape(NR, L), gk_h, dst_h, s_h, z02)
  return jnp.concatenate([z2[0], z2[1]], axis=1)
